# baseline (speedup 1.0000x reference)
import jax
from jax.experimental import pallas as pl
from jax.experimental.pallas import tpu as pltpu

_CHUNKS = 16
_BUFS = 8


def _copy_kernel(in_hbm, out_hbm, bufs, in_sems, out_sems):
    def in_copy(i):
        s = i % _BUFS
        return pltpu.make_async_copy(in_hbm.at[i], bufs.at[s], in_sems.at[s])

    def out_copy(i):
        s = i % _BUFS
        return pltpu.make_async_copy(bufs.at[s], out_hbm.at[i], out_sems.at[s])

    for i in range(_BUFS):
        in_copy(i).start()
    for i in range(_CHUNKS):
        if i >= _BUFS:
            out_copy(i - _BUFS).wait()
            in_copy(i).start()
        in_copy(i).wait()
        out_copy(i).start()
    for i in range(_CHUNKS - _BUFS, _CHUNKS):
        out_copy(i).wait()


def kernel(image_token, text_cls, topk_idx, selected_pooled, is_rare, strength):
    B, N, D = image_token.shape
    rows = (B * N) // _CHUNKS
    x = image_token.reshape(_CHUNKS, rows, D)
    out = pl.pallas_call(
        _copy_kernel,
        out_shape=jax.ShapeDtypeStruct(x.shape, x.dtype),
        in_specs=[pl.BlockSpec(memory_space=pl.ANY)],
        out_specs=pl.BlockSpec(memory_space=pl.ANY),
        scratch_shapes=[
            pltpu.VMEM((_BUFS, rows, D), x.dtype),
            pltpu.SemaphoreType.DMA((_BUFS,)),
            pltpu.SemaphoreType.DMA((_BUFS,)),
        ],
    )(x)
    return out.reshape(B, N, D)
